# Initial kernel scaffold; baseline (speedup 1.0000x reference)
import jax
import jax.numpy as jnp
from jax.experimental import pallas as pl


def kernel(x, activations, W1, W2):
    raise NotImplementedError("write your pallas kernel here")


# speedup vs baseline: 1.4300x; 1.4300x over previous
import functools

import jax
import jax.numpy as jnp
from jax import lax
from jax.experimental import pallas as pl
from jax.experimental.pallas import tpu as pltpu
from jax.experimental.pallas import tpu_sc as plsc

N = 8192
D = 2048
DFF = 8192
KEEP = 4096

_RB = 128


def _rank_body(acol_ref, arow_ref, rank_ref):
    ai = acol_ref[...]
    aj = arow_ref[...]
    i0 = pl.program_id(0) * _RB
    ii = i0 + lax.broadcasted_iota(jnp.int32, (_RB, N), 0)
    jj = lax.broadcasted_iota(jnp.int32, (_RB, N), 1)
    before = (aj > ai) | ((aj == ai) & (jj < ii))
    rank_ref[...] = jnp.sum(before.astype(jnp.int32), axis=1).reshape(1, 1, _RB)


def _compute_ranks(a):
    acol = a.reshape(N, 1)
    arow = a.reshape(1, N)
    ranks = pl.pallas_call(
        _rank_body,
        grid=(N // _RB,),
        in_specs=[
            pl.BlockSpec((_RB, 1), lambda i: (i, 0)),
            pl.BlockSpec((1, N), lambda i: (0, 0)),
        ],
        out_specs=pl.BlockSpec((1, 1, _RB), lambda i: (i, 0, 0)),
        out_shape=jax.ShapeDtypeStruct((N // _RB, 1, _RB), jnp.int32),
    )(acol, arow)
    return ranks.reshape(1, N)


def _invert_body(ranks_ref, idx_ref):
    rr = ranks_ref[...]
    p0 = pl.program_id(0) * _RB
    pp = p0 + lax.broadcasted_iota(jnp.int32, (_RB, N), 0)
    ii = lax.broadcasted_iota(jnp.int32, (_RB, N), 1)
    hit = jnp.where(rr == pp, ii, 0)
    idx_ref[...] = jnp.sum(hit, axis=1).reshape(1, 1, _RB)


def _invert_ranks(ranks):
    idx = pl.pallas_call(
        _invert_body,
        grid=(KEEP // _RB,),
        in_specs=[pl.BlockSpec((1, N), lambda p: (0, 0))],
        out_specs=pl.BlockSpec((1, 1, _RB), lambda p: (p, 0, 0)),
        out_shape=jax.ShapeDtypeStruct((KEEP // _RB, 1, _RB), jnp.int32),
    )(ranks)
    return idx.reshape(KEEP)


_NC = 2
_NS = 16
_NW = _NC * _NS
_RPW = KEEP // _NW
_CH = 16
_NCH = _RPW // _CH


def _sc_gather_body(idx_hbm, x_hbm, out_hbm, idx_v, buf0, buf1, sem0, sem1):
    wid = lax.axis_index("s") * _NC + lax.axis_index("c")
    lo = wid * _RPW

    pltpu.sync_copy(idx_hbm.at[pl.ds(lo, _RPW)], idx_v)

    bufs = (buf0, buf1)
    sems = (sem0, sem1)
    copies = [None, None]
    copies[0] = pltpu.async_copy(x_hbm.at[idx_v.at[pl.ds(0, _CH)]], bufs[0], sems[0])
    for c in range(_NCH):
        if c + 1 < _NCH:
            copies[(c + 1) % 2] = pltpu.async_copy(
                x_hbm.at[idx_v.at[pl.ds((c + 1) * _CH, _CH)]],
                bufs[(c + 1) % 2],
                sems[(c + 1) % 2],
            )
        copies[c % 2].wait()
        pltpu.sync_copy(bufs[c % 2], out_hbm.at[pl.ds(lo + c * _CH, _CH)])


@functools.cache
def _sc_gather_kernel():
    mesh = plsc.VectorSubcoreMesh(
        core_axis_name="c", subcore_axis_name="s", num_cores=_NC, num_subcores=_NS
    )
    return pl.kernel(
        _sc_gather_body,
        out_type=jax.ShapeDtypeStruct((KEEP, D), jnp.float32),
        mesh=mesh,
        scratch_types=[
            pltpu.VMEM((_RPW,), jnp.int32),
            pltpu.VMEM((_CH, D), jnp.float32),
            pltpu.VMEM((_CH, D), jnp.float32),
            pltpu.SemaphoreType.DMA,
            pltpu.SemaphoreType.DMA,
        ],
    )


def _sc_gather(top_idx, x):
    return _sc_gather_kernel()(top_idx, x)


def _mm1_body(a_ref, b_ref, o_ref):
    o_ref[...] = jnp.maximum(
        jnp.dot(a_ref[...], b_ref[...], preferred_element_type=jnp.float32), 0.0
    )


def _mm1(xg, W1):
    bm, bn = 512, 2048
    return pl.pallas_call(
        _mm1_body,
        grid=(DFF // bn, KEEP // bm),
        in_specs=[
            pl.BlockSpec((bm, D), lambda n, m: (m, 0)),
            pl.BlockSpec((D, bn), lambda n, m: (0, n)),
        ],
        out_specs=pl.BlockSpec((bm, bn), lambda n, m: (m, n)),
        out_shape=jax.ShapeDtypeStruct((KEEP, DFF), jnp.float32),
    )(xg, W1)


def _mm2_body(a_ref, b_ref, o_ref):
    @pl.when(pl.program_id(0) == 0)
    def _init():
        o_ref[...] = jnp.zeros_like(o_ref)

    o_ref[...] += jnp.dot(a_ref[...], b_ref[...], preferred_element_type=jnp.float32)


def _mm2(h, W2):
    bk = 256
    return pl.pallas_call(
        _mm2_body,
        grid=(DFF // bk,),
        in_specs=[
            pl.BlockSpec((KEEP, bk), lambda k: (0, k)),
            pl.BlockSpec((bk, D), lambda k: (k, 0)),
        ],
        out_specs=pl.BlockSpec((KEEP, D), lambda k: (0, 0)),
        out_shape=jax.ShapeDtypeStruct((KEEP, D), jnp.float32),
    )(h, W2)


def kernel(x, activations, W1, W2):
    ranks = _compute_ranks(activations)
    top_idx = _invert_ranks(ranks)
    xg = _sc_gather(top_idx, x)
    h = _mm1(xg, W1)
    return _mm2(h, W2)
